# baseline (speedup 1.0000x reference)
import functools

import jax
import jax.numpy as jnp
from jax import lax
from jax.experimental import pallas as pl
from jax.experimental.pallas import tpu as pltpu
from jax.experimental.pallas import tpu_sc as plsc

N_NEURONS = 64
TABLE_SIZE = 65536
BATCH = 4096

NUM_CORES = 2
NUM_SUBCORES = 16
LANES = 16
NUM_WORKERS = NUM_CORES * NUM_SUBCORES
BPW = BATCH // NUM_WORKERS
JV = BPW // LANES


ROWS_PER_CHUNK = 4
CHUNK = ROWS_PER_CHUNK * BPW
N_CHUNKS = N_NEURONS // ROWS_PER_CHUNK


def _sc_body(data_hbm, sums_hbm, counts_hbm, out_hbm,
             st_v, idx_v, gs_v, gc_v, out_v, sem_s, sem_c):
    c = lax.axis_index("c")
    s = lax.axis_index("s")
    wid = s * NUM_CORES + c
    base = wid * BPW

    pltpu.sync_copy(data_hbm.at[:, pl.ds(base, BPW)], st_v)

    def fire_sums(k, carry):
        for r in range(ROWS_PER_CHUNK):
            n = k * ROWS_PER_CHUNK + r
            row_base = (n // 8) * (8 * TABLE_SIZE) + (n % 8) * 128
            for j in range(JV):
                a = st_v[n, pl.ds(j * LANES, LANES)]
                idx_v[pl.ds(n * BPW + j * LANES, LANES)] = (
                    ((a >> 7) << 10) + (a & 127)
                    + jnp.full((LANES,), row_base, dtype=jnp.int32)
                )
        sl = pl.ds(k * CHUNK, CHUNK)
        pltpu.async_copy(sums_hbm.at[idx_v.at[sl]], gs_v.at[sl], sem_s)
        return carry

    def fire_counts(k, carry):
        sl = pl.ds(k * CHUNK, CHUNK)
        pltpu.async_copy(counts_hbm.at[idx_v.at[sl]], gc_v.at[sl], sem_c)
        return carry

    lax.fori_loop(0, N_CHUNKS, fire_sums, 0)
    lax.fori_loop(0, N_CHUNKS, fire_counts, 0)

    def drain(tbl_hbm, g_v, sem):
        def body(k, carry):
            sl = pl.ds(k * CHUNK, CHUNK)
            pltpu.make_async_copy(
                tbl_hbm.at[idx_v.at[sl]], g_v.at[sl], sem).wait()
            return carry
        lax.fori_loop(0, N_CHUNKS, body, 0)

    def accumulate(g_ref):
        def body(n, carry):
            return tuple(
                carry[j] + g_ref[pl.ds(n * BPW + j * LANES, LANES)]
                for j in range(JV)
            )
        zero = tuple(jnp.zeros((LANES,), jnp.float32) for _ in range(JV))
        return lax.fori_loop(0, N_NEURONS, body, zero)

    drain(sums_hbm, gs_v, sem_s)
    resp = accumulate(gs_v)
    drain(counts_hbm, gc_v, sem_c)
    cnt = accumulate(gc_v)

    for j in range(JV):
        safe = cnt[j] > 0.0
        denom = jnp.where(safe, cnt[j], jnp.ones((LANES,), jnp.float32))
        out_v[pl.ds(j * LANES, LANES)] = jnp.where(
            safe, resp[j] / denom, jnp.zeros((LANES,), jnp.float32))
    pltpu.sync_copy(out_v, out_hbm.at[pl.ds(base, BPW)])


@functools.partial(
    pl.kernel,
    out_type=jax.ShapeDtypeStruct((BATCH,), jnp.float32),
    mesh=plsc.VectorSubcoreMesh(core_axis_name="c", subcore_axis_name="s"),
    scratch_types=[
        pltpu.VMEM((N_NEURONS, BPW), jnp.int32),
        pltpu.VMEM((N_NEURONS * BPW,), jnp.int32),
        pltpu.VMEM((N_NEURONS * BPW,), jnp.float32),
        pltpu.VMEM((N_NEURONS * BPW,), jnp.float32),
        pltpu.VMEM((BPW,), jnp.float32),
        pltpu.SemaphoreType.DMA,
        pltpu.SemaphoreType.DMA,
    ],
)
def _wisard_sc(*refs):
    _sc_body(*refs)


def _as_linear(x):
    return x.reshape(8, 8, 512, 128).transpose(0, 2, 1, 3).reshape(-1)


def kernel(data, sums, counts):
    return _wisard_sc(data, _as_linear(sums), _as_linear(counts))


# speedup vs baseline: 1.0363x; 1.0363x over previous
import functools

import jax
import jax.numpy as jnp
from jax import lax
from jax.experimental import pallas as pl
from jax.experimental.pallas import tpu as pltpu
from jax.experimental.pallas import tpu_sc as plsc

N_NEURONS = 64
TABLE_SIZE = 65536
BATCH = 4096

NUM_CORES = 2
NUM_SUBCORES = 16
LANES = 16
NUM_WORKERS = NUM_CORES * NUM_SUBCORES
BPW = BATCH // NUM_WORKERS
JV = BPW // LANES


ROWS_PER_CHUNK = 4
CHUNK = ROWS_PER_CHUNK * BPW
N_CHUNKS = N_NEURONS // ROWS_PER_CHUNK


def _sc_body(data_hbm, sums_hbm, counts_hbm, out_hbm,
             st_v, idx_v, gs_v, gc_v, out_v, sem_s, sem_c):
    c = lax.axis_index("c")
    s = lax.axis_index("s")
    wid = s * NUM_CORES + c
    base = wid * BPW

    pltpu.sync_copy(data_hbm.at[:, pl.ds(base, BPW)], st_v)

    def fire(k, carry):
        for r in range(ROWS_PER_CHUNK):
            n = k * ROWS_PER_CHUNK + r
            row_base = (n // 8) * (8 * TABLE_SIZE) + (n % 8) * 128
            for j in range(JV):
                a = st_v[n, pl.ds(j * LANES, LANES)]
                idx_v[pl.ds(n * BPW + j * LANES, LANES)] = (
                    ((a >> 7) << 10) + (a & 127)
                    + jnp.full((LANES,), row_base, dtype=jnp.int32)
                )
        sl = pl.ds(k * CHUNK, CHUNK)
        pltpu.async_copy(sums_hbm.at[idx_v.at[sl]], gs_v.at[sl], sem_s)
        pltpu.async_copy(counts_hbm.at[idx_v.at[sl]], gc_v.at[sl], sem_c)
        return carry

    lax.fori_loop(0, N_CHUNKS, fire, 0)

    def drain(tbl_hbm, g_v, sem):
        def body(k, carry):
            sl = pl.ds(k * CHUNK, CHUNK)
            pltpu.make_async_copy(
                tbl_hbm.at[idx_v.at[sl]], g_v.at[sl], sem).wait()
            return carry
        lax.fori_loop(0, N_CHUNKS, body, 0)

    def accumulate(g_ref):
        def body(n, carry):
            return tuple(
                carry[j] + g_ref[pl.ds(n * BPW + j * LANES, LANES)]
                for j in range(JV)
            )
        zero = tuple(jnp.zeros((LANES,), jnp.float32) for _ in range(JV))
        return lax.fori_loop(0, N_NEURONS, body, zero)

    drain(sums_hbm, gs_v, sem_s)
    resp = accumulate(gs_v)
    drain(counts_hbm, gc_v, sem_c)
    cnt = accumulate(gc_v)

    for j in range(JV):
        safe = cnt[j] > 0.0
        denom = jnp.where(safe, cnt[j], jnp.ones((LANES,), jnp.float32))
        out_v[pl.ds(j * LANES, LANES)] = jnp.where(
            safe, resp[j] / denom, jnp.zeros((LANES,), jnp.float32))
    pltpu.sync_copy(out_v, out_hbm.at[pl.ds(base, BPW)])


@functools.partial(
    pl.kernel,
    out_type=jax.ShapeDtypeStruct((BATCH,), jnp.float32),
    mesh=plsc.VectorSubcoreMesh(core_axis_name="c", subcore_axis_name="s"),
    scratch_types=[
        pltpu.VMEM((N_NEURONS, BPW), jnp.int32),
        pltpu.VMEM((N_NEURONS * BPW,), jnp.int32),
        pltpu.VMEM((N_NEURONS * BPW,), jnp.float32),
        pltpu.VMEM((N_NEURONS * BPW,), jnp.float32),
        pltpu.VMEM((BPW,), jnp.float32),
        pltpu.SemaphoreType.DMA,
        pltpu.SemaphoreType.DMA,
    ],
)
def _wisard_sc(*refs):
    _sc_body(*refs)


def _as_linear(x):
    return x.reshape(8, 8, 512, 128).transpose(0, 2, 1, 3).reshape(-1)


def kernel(data, sums, counts):
    return _wisard_sc(data, _as_linear(sums), _as_linear(counts))
